# baseline (speedup 1.0000x reference)
import jax
import jax.numpy as jnp
from jax.experimental import pallas as pl

_BR = 200


def _xw_body(x_ref, w_ref, o_ref):
    o_ref[...] = jnp.dot(x_ref[...].astype(jnp.bfloat16),
                         w_ref[...].astype(jnp.bfloat16),
                         preferred_element_type=jnp.float32)


def _l1_body(adj_ref, s_ref, b_ref, w2_ref, w3_ref, o_ref):
    h = jnp.dot(adj_ref[...].astype(jnp.bfloat16),
                s_ref[...].astype(jnp.bfloat16),
                preferred_element_type=jnp.float32)
    h = jnp.maximum(h + b_ref[...], 0.0)
    hw2 = jnp.dot(h, w2_ref[...], preferred_element_type=jnp.float32)
    o_ref[...] = jnp.dot(hw2, w3_ref[...], preferred_element_type=jnp.float32)


def _l2_body(adj_ref, s_ref, b2_ref, w3_ref, o_ref):
    c = jnp.dot(b2_ref[...], w3_ref[...], preferred_element_type=jnp.float32)
    o_ref[...] = jnp.dot(adj_ref[...].astype(jnp.bfloat16),
                         s_ref[...].astype(jnp.bfloat16),
                         preferred_element_type=jnp.float32) + c


def _l3_body(adj_ref, s_ref, b_ref, o_ref):
    h = jnp.dot(adj_ref[...].astype(jnp.bfloat16),
                s_ref[...].astype(jnp.bfloat16),
                preferred_element_type=jnp.float32) + b_ref[...]
    m = jnp.max(h, axis=1, keepdims=True)
    lse = jnp.log(jnp.sum(jnp.exp(h - m), axis=1, keepdims=True))
    o_ref[...] = h - m - lse


def kernel(x, adj, W1, b1, W2, b2, W3, b3):
    N, F = x.shape
    d1 = W1.shape[1]
    d2 = W2.shape[1]
    d3 = W3.shape[1]
    b1r = b1.reshape(1, d1)
    b2r = b2.reshape(1, d2)
    b3r = b3.reshape(1, d3)

    grid = (N // _BR,)
    row = lambda i: (i, 0)
    const = lambda i: (0, 0)

    s1 = pl.pallas_call(
        _xw_body,
        grid=grid,
        in_specs=[pl.BlockSpec((_BR, F), row),
                  pl.BlockSpec((F, d1), const)],
        out_specs=pl.BlockSpec((_BR, d1), row),
        out_shape=jax.ShapeDtypeStruct((N, d1), jnp.float32),
    )(x, W1)

    u = pl.pallas_call(
        _l1_body,
        grid=grid,
        in_specs=[pl.BlockSpec((_BR, N), row),
                  pl.BlockSpec((N, d1), const),
                  pl.BlockSpec((1, d1), const),
                  pl.BlockSpec((d1, d2), const),
                  pl.BlockSpec((d2, d3), const)],
        out_specs=pl.BlockSpec((_BR, d3), row),
        out_shape=jax.ShapeDtypeStruct((N, d3), jnp.float32),
    )(adj, s1, b1r, W2, W3)

    t = pl.pallas_call(
        _l2_body,
        grid=grid,
        in_specs=[pl.BlockSpec((_BR, N), row),
                  pl.BlockSpec((N, d3), const),
                  pl.BlockSpec((1, d2), const),
                  pl.BlockSpec((d2, d3), const)],
        out_specs=pl.BlockSpec((_BR, d3), row),
        out_shape=jax.ShapeDtypeStruct((N, d3), jnp.float32),
    )(adj, u, b2r, W3)

    out = pl.pallas_call(
        _l3_body,
        grid=grid,
        in_specs=[pl.BlockSpec((_BR, N), row),
                  pl.BlockSpec((N, d3), const),
                  pl.BlockSpec((1, d3), const)],
        out_specs=pl.BlockSpec((_BR, d3), row),
        out_shape=jax.ShapeDtypeStruct((N, d3), jnp.float32),
    )(adj, t, b3r)
    return out


# speedup vs baseline: 1.0402x; 1.0402x over previous
import functools

import jax
import jax.numpy as jnp
from jax.experimental import pallas as pl
from jax.experimental.pallas import tpu as pltpu

_BR = 200
_K = 4


def _xw_body(x_ref, w_ref, o_ref):
    o_ref[...] = jnp.dot(x_ref[...].astype(jnp.bfloat16),
                         w_ref[...].astype(jnp.bfloat16),
                         preferred_element_type=jnp.float32)


def _adj_body(epilogue, n_extra, adj_hbm, s_ref, *rest):
    extras = rest[:n_extra]
    o_ref = rest[n_extra]
    abuf = rest[n_extra + 1]
    sems = rest[n_extra + 2]
    i = pl.program_id(0)
    nstep = pl.num_programs(0)

    @pl.when(i == 0)
    def _prologue():
        for k in range(_K - 1):
            pltpu.make_async_copy(adj_hbm.at[pl.ds(k * _BR, _BR), :],
                                  abuf.at[k], sems.at[k]).start()

    nxt = i + _K - 1

    @pl.when(nxt < nstep)
    def _issue_ahead():
        b = jax.lax.rem(nxt, _K)
        pltpu.make_async_copy(adj_hbm.at[pl.ds(nxt * _BR, _BR), :],
                              abuf.at[b], sems.at[b]).start()

    b = jax.lax.rem(i, _K)
    pltpu.make_async_copy(adj_hbm.at[pl.ds(i * _BR, _BR), :],
                          abuf.at[b], sems.at[b]).wait()
    h = jnp.dot(abuf[b].astype(jnp.bfloat16),
                s_ref[...].astype(jnp.bfloat16),
                preferred_element_type=jnp.float32)
    o_ref[...] = epilogue(h, *[e[...] for e in extras])


def _ep_l1(h, b1, w2, w3):
    h = jnp.maximum(h + b1, 0.0)
    return jnp.dot(jnp.dot(h, w2, preferred_element_type=jnp.float32),
                   w3, preferred_element_type=jnp.float32)


def _ep_l2(h, b2, w3):
    return h + jnp.dot(b2, w3, preferred_element_type=jnp.float32)


def _ep_l3(h, b3):
    h = h + b3
    m = jnp.max(h, axis=1, keepdims=True)
    lse = jnp.log(jnp.sum(jnp.exp(h - m), axis=1, keepdims=True))
    return h - m - lse


def _adj_pass(adj, s, extras, epilogue, d_out):
    N = adj.shape[0]
    d_s = s.shape[1]
    const = lambda i: (0, 0)
    in_specs = [pl.BlockSpec(memory_space=pltpu.HBM),
                pl.BlockSpec((N, d_s), const)]
    for e in extras:
        in_specs.append(pl.BlockSpec(e.shape, const))
    return pl.pallas_call(
        functools.partial(_adj_body, epilogue, len(extras)),
        grid=(N // _BR,),
        in_specs=in_specs,
        out_specs=pl.BlockSpec((_BR, d_out), lambda i: (i, 0)),
        out_shape=jax.ShapeDtypeStruct((N, d_out), jnp.float32),
        scratch_shapes=[pltpu.VMEM((_K, _BR, N), jnp.float32),
                        pltpu.SemaphoreType.DMA((_K,))],
    )(adj, s, *extras)


def kernel(x, adj, W1, b1, W2, b2, W3, b3):
    N, F = x.shape
    d1 = W1.shape[1]
    d2 = W2.shape[1]
    d3 = W3.shape[1]
    b1r = b1.reshape(1, d1)
    b2r = b2.reshape(1, d2)
    b3r = b3.reshape(1, d3)

    BRX = 400
    s1 = pl.pallas_call(
        _xw_body,
        grid=(N // BRX,),
        in_specs=[pl.BlockSpec((BRX, F), lambda i: (i, 0)),
                  pl.BlockSpec((F, d1), lambda i: (0, 0))],
        out_specs=pl.BlockSpec((BRX, d1), lambda i: (i, 0)),
        out_shape=jax.ShapeDtypeStruct((N, d1), jnp.float32),
    )(x, W1)

    u = _adj_pass(adj, s1, (b1r, W2, W3), _ep_l1, d3)
    t = _adj_pass(adj, u, (b2r, W3), _ep_l2, d3)
    return _adj_pass(adj, t, (b3r,), _ep_l3, d3)


# speedup vs baseline: 1.1693x; 1.1241x over previous
import jax
import jax.numpy as jnp
from jax.experimental import pallas as pl

_BR = 400


def _xw_body(x_ref, w_ref, o_ref):
    o_ref[...] = jnp.dot(x_ref[...].astype(jnp.bfloat16),
                         w_ref[...].astype(jnp.bfloat16),
                         preferred_element_type=jnp.float32)


def _l1_body(adj_ref, s_ref, b_ref, w2_ref, w3_ref, o_ref, a16_ref):
    a16 = adj_ref[...].astype(jnp.bfloat16)
    a16_ref[...] = a16
    h = jnp.dot(a16, s_ref[...].astype(jnp.bfloat16),
                preferred_element_type=jnp.float32)
    h = jnp.maximum(h + b_ref[...], 0.0)
    hw2 = jnp.dot(h, w2_ref[...], preferred_element_type=jnp.float32)
    o_ref[...] = jnp.dot(hw2, w3_ref[...], preferred_element_type=jnp.float32)


def _l2_body(a16_ref, s_ref, b2_ref, w3_ref, o_ref):
    c = jnp.dot(b2_ref[...], w3_ref[...], preferred_element_type=jnp.float32)
    o_ref[...] = jnp.dot(a16_ref[...], s_ref[...].astype(jnp.bfloat16),
                         preferred_element_type=jnp.float32) + c


def _l3_body(a16_ref, s_ref, b_ref, o_ref):
    h = jnp.dot(a16_ref[...], s_ref[...].astype(jnp.bfloat16),
                preferred_element_type=jnp.float32) + b_ref[...]
    m = jnp.max(h, axis=1, keepdims=True)
    lse = jnp.log(jnp.sum(jnp.exp(h - m), axis=1, keepdims=True))
    o_ref[...] = h - m - lse


def kernel(x, adj, W1, b1, W2, b2, W3, b3):
    N, F = x.shape
    d1 = W1.shape[1]
    d2 = W2.shape[1]
    d3 = W3.shape[1]
    b1r = b1.reshape(1, d1)
    b2r = b2.reshape(1, d2)
    b3r = b3.reshape(1, d3)

    grid = (N // _BR,)
    row = lambda i: (i, 0)
    const = lambda i: (0, 0)

    s1 = pl.pallas_call(
        _xw_body,
        grid=grid,
        in_specs=[pl.BlockSpec((_BR, F), row),
                  pl.BlockSpec((F, d1), const)],
        out_specs=pl.BlockSpec((_BR, d1), row),
        out_shape=jax.ShapeDtypeStruct((N, d1), jnp.float32),
    )(x, W1)

    u, adj16 = pl.pallas_call(
        _l1_body,
        grid=grid,
        in_specs=[pl.BlockSpec((_BR, N), row),
                  pl.BlockSpec((N, d1), const),
                  pl.BlockSpec((1, d1), const),
                  pl.BlockSpec((d1, d2), const),
                  pl.BlockSpec((d2, d3), const)],
        out_specs=[pl.BlockSpec((_BR, d3), row),
                   pl.BlockSpec((_BR, N), row)],
        out_shape=[jax.ShapeDtypeStruct((N, d3), jnp.float32),
                   jax.ShapeDtypeStruct((N, N), jnp.bfloat16)],
    )(adj, s1, b1r, W2, W3)

    t = pl.pallas_call(
        _l2_body,
        grid=grid,
        in_specs=[pl.BlockSpec((_BR, N), row),
                  pl.BlockSpec((N, d3), const),
                  pl.BlockSpec((1, d2), const),
                  pl.BlockSpec((d2, d3), const)],
        out_specs=pl.BlockSpec((_BR, d3), row),
        out_shape=jax.ShapeDtypeStruct((N, d3), jnp.float32),
    )(adj16, u, b2r, W3)

    return pl.pallas_call(
        _l3_body,
        grid=grid,
        in_specs=[pl.BlockSpec((_BR, N), row),
                  pl.BlockSpec((N, d3), const),
                  pl.BlockSpec((1, d3), const)],
        out_specs=pl.BlockSpec((_BR, d3), row),
        out_shape=jax.ShapeDtypeStruct((N, d3), jnp.float32),
    )(adj16, t, b3r)
